# Initial kernel scaffold; baseline (speedup 1.0000x reference)
import jax
import jax.numpy as jnp
from jax.experimental import pallas as pl


def kernel(x, edge_index, W1, b1, W2, b2, W_ih, W_hh, b_ih, b_hh, Wfc, bfc):
    raise NotImplementedError("write your pallas kernel here")


# speedup vs baseline: 4.3378x; 4.3378x over previous
import functools
import jax
import jax.numpy as jnp
from jax import lax
from jax.experimental import pallas as pl
from jax.experimental.pallas import tpu as pltpu
from jax.experimental.pallas import tpu_sc as plsc

N = 10000
E = 320000
F = 128
DEGW = 16
NC, NS = 2, 16
NW = NC * NS
EPW = E // NW
CH = 128
NFULL = EPW // CH
REM = EPW - NFULL * CH
RPS = 624
TAIL = N - NS * RPS

_mesh = plsc.VectorSubcoreMesh(core_axis_name="c", subcore_axis_name="s")


@functools.partial(
    pl.kernel,
    out_type=jax.ShapeDtypeStruct((NC, N, F), jnp.float32),
    mesh=_mesh,
    scratch_types=[
        pltpu.VMEM((CH,), jnp.int32),
        pltpu.VMEM((REM,), jnp.int32),
        pltpu.VMEM((CH, F), jnp.float32),
        pltpu.VMEM_SHARED((N, F), jnp.float32),
        pltpu.SemaphoreType.DMA,
    ],
)
def _sc_degree(dst_hbm, ones_hbm, z_hbm, out_hbm, didx, didx2, ones_v, acc,
               sem):
    c = lax.axis_index("c")
    s = lax.axis_index("s")
    wid = s * NC + c
    r0 = s * RPS
    pltpu.sync_copy(z_hbm.at[pl.ds(r0, RPS)], acc.at[pl.ds(r0, RPS)])

    @pl.when(s == NS - 1)
    def _():
        pltpu.sync_copy(z_hbm.at[pl.ds(NS * RPS, TAIL)],
                        acc.at[pl.ds(NS * RPS, TAIL)])

    pltpu.sync_copy(ones_hbm, ones_v)
    plsc.subcore_barrier()
    base = wid * EPW

    def chunk(i, carry):
        off = base + i * CH
        pltpu.sync_copy(dst_hbm.at[pl.ds(off, CH)], didx)
        pltpu.sync_copy(ones_v, acc.at[didx], add=True)
        return carry

    lax.fori_loop(0, NFULL, chunk, 0)
    off = base + NFULL * CH
    pltpu.sync_copy(dst_hbm.at[pl.ds(off, REM)], didx2)
    pltpu.sync_copy(ones_v.at[pl.ds(0, REM)], acc.at[didx2], add=True)
    plsc.subcore_barrier()
    pltpu.sync_copy(acc.at[pl.ds(r0, RPS)], out_hbm.at[c, pl.ds(r0, RPS)])

    @pl.when(s == NS - 1)
    def _():
        pltpu.sync_copy(acc.at[pl.ds(NS * RPS, TAIL)],
                        out_hbm.at[c, pl.ds(NS * RPS, TAIL)])


@functools.partial(
    pl.kernel,
    out_type=jax.ShapeDtypeStruct((NC, N, F), jnp.float32),
    mesh=_mesh,
    scratch_types=[
        pltpu.VMEM((CH,), jnp.int32),
        pltpu.VMEM((CH,), jnp.int32),
        pltpu.VMEM((CH, F), jnp.float32),
        pltpu.VMEM((REM,), jnp.int32),
        pltpu.VMEM((REM,), jnp.int32),
        pltpu.VMEM((REM, F), jnp.float32),
        pltpu.VMEM_SHARED((N, F), jnp.float32),
        pltpu.SemaphoreType.DMA,
    ],
)
def _sc_scatter(y_hbm, src_hbm, dst_hbm, z_hbm, out_hbm,
                sidx, didx, rows, sidx2, didx2, rows2, acc, sem):
    c = lax.axis_index("c")
    s = lax.axis_index("s")
    wid = s * NC + c
    r0 = s * RPS
    pltpu.sync_copy(z_hbm.at[pl.ds(r0, RPS)], acc.at[pl.ds(r0, RPS)])

    @pl.when(s == NS - 1)
    def _():
        pltpu.sync_copy(z_hbm.at[pl.ds(NS * RPS, TAIL)],
                        acc.at[pl.ds(NS * RPS, TAIL)])

    plsc.subcore_barrier()
    base = wid * EPW

    def chunk(i, carry):
        off = base + i * CH
        pltpu.sync_copy(src_hbm.at[pl.ds(off, CH)], sidx)
        pltpu.sync_copy(dst_hbm.at[pl.ds(off, CH)], didx)
        pltpu.async_copy(y_hbm.at[sidx], rows, sem).wait()
        pltpu.sync_copy(rows, acc.at[didx], add=True)
        return carry

    lax.fori_loop(0, NFULL, chunk, 0)
    off = base + NFULL * CH
    pltpu.sync_copy(src_hbm.at[pl.ds(off, REM)], sidx2)
    pltpu.sync_copy(dst_hbm.at[pl.ds(off, REM)], didx2)
    pltpu.async_copy(y_hbm.at[sidx2], rows2, sem).wait()
    pltpu.sync_copy(rows2, acc.at[didx2], add=True)
    plsc.subcore_barrier()
    pltpu.sync_copy(acc.at[pl.ds(r0, RPS)], out_hbm.at[c, pl.ds(r0, RPS)])

    @pl.when(s == NS - 1)
    def _():
        pltpu.sync_copy(acc.at[pl.ds(NS * RPS, TAIL)],
                        out_hbm.at[c, pl.ds(NS * RPS, TAIL)])


RB = 2000


def _k1_body(x_ref, w_ref, dinv_ref, xlin_ref, y_ref):
    xl = jnp.dot(x_ref[...], w_ref[...], preferred_element_type=jnp.float32)
    xlin_ref[...] = xl
    y_ref[...] = xl * dinv_ref[...]


def _k2_body(a_ref, b_ref, xlin_ref, dinv_ref, bias_ref, w_ref,
             xlin2_ref, y2_ref):
    dv = dinv_ref[...]
    agg = (a_ref[...] + b_ref[...]) * dv + xlin_ref[...] * (dv * dv)
    h = jnp.maximum(agg + bias_ref[...], 0.0)
    xl2 = jnp.dot(h, w_ref[...], preferred_element_type=jnp.float32)
    xlin2_ref[...] = xl2
    y2_ref[...] = xl2 * dv


def _k3_body(a_ref, b_ref, xlin_ref, dinv_ref, bias_ref, w_ref, gb_ref,
             gx_ref):
    dv = dinv_ref[...]
    agg = (a_ref[...] + b_ref[...]) * dv + xlin_ref[...] * (dv * dv)
    h = jnp.maximum(agg + bias_ref[...], 0.0)
    gx_ref[...] = (jnp.dot(h, w_ref[...], preferred_element_type=jnp.float32)
                   + gb_ref[...])


def _row_spec(w):
    return pl.BlockSpec((RB, w), lambda i: (i, 0))


def _full_spec(shape):
    return pl.BlockSpec(shape, lambda i: tuple(0 for _ in shape))


def _lstm_body(gx_ref, whh_ref, wfc_ref, bfc_ref, out_ref):
    H = 32

    def step(t, carry):
        h128, c = carry
        hw = jnp.dot(h128, whh_ref[...], preferred_element_type=jnp.float32)
        gates = gx_ref[pl.ds(t, 1), :] + hw
        sg = jax.nn.sigmoid(gates)
        th = jnp.tanh(gates)
        i = sg[:, 0:H]
        f = sg[:, H:2 * H]
        g = th[:, 2 * H:3 * H]
        o = sg[:, 3 * H:4 * H]
        c2 = f * c + i * g
        h32 = o * jnp.tanh(c2)
        h128n = jnp.concatenate(
            [h32, jnp.zeros((1, F - H), jnp.float32)], axis=1)
        return (h128n, c2)

    h0 = jnp.zeros((1, F), jnp.float32)
    c0 = jnp.zeros((1, H), jnp.float32)
    hN, _ = lax.fori_loop(0, N, step, (h0, c0))
    out_ref[...] = (jnp.sum(hN * wfc_ref[...]) + bfc_ref[0, 0]).reshape(1, 1)


def _pad2(a, r, c):
    return jnp.zeros((r, c), jnp.float32).at[:a.shape[0], :a.shape[1]].set(a)


@jax.jit
def kernel(x, edge_index, W1, b1, W2, b2, W_ih, W_hh, b_ih, b_hh, Wfc, bfc):
    src = edge_index[0].astype(jnp.int32)
    dst = edge_index[1].astype(jnp.int32)

    z128 = jnp.zeros((N, F), jnp.float32)
    ones = jnp.ones((CH, F), jnp.float32)

    W1p = _pad2(W1, F, F)
    W2p = _pad2(W2, F, F)
    Wihp = _pad2(W_ih.T, F, F)
    whh_big = _pad2(W_hh.T, F, F)
    b1p = _pad2(b1[None, :], 1, F)
    b2p = _pad2(b2[None, :], 1, F)
    gbias = _pad2((b_ih + b_hh)[None, :], 1, F)
    wfc_pad = _pad2(Wfc.T, 1, F)
    bfc2d = bfc.reshape(1, 1)

    deg_parts = _sc_degree(dst, ones, z128)
    deg = deg_parts[0, :, 0] + deg_parts[1, :, 0] + 1.0
    dinv2d = lax.rsqrt(deg)[:, None]

    grid = (N // RB,)
    xlin1, y1 = pl.pallas_call(
        _k1_body,
        grid=grid,
        in_specs=[_row_spec(F), _full_spec((F, F)), _row_spec(1)],
        out_specs=[_row_spec(F), _row_spec(F)],
        out_shape=[jax.ShapeDtypeStruct((N, F), jnp.float32)] * 2,
    )(x, W1p, dinv2d)

    acc1 = _sc_scatter(y1, src, dst, z128)

    xlin2, y2 = pl.pallas_call(
        _k2_body,
        grid=grid,
        in_specs=[_row_spec(F), _row_spec(F), _row_spec(F), _row_spec(1),
                  _full_spec((1, F)), _full_spec((F, F))],
        out_specs=[_row_spec(F), _row_spec(F)],
        out_shape=[jax.ShapeDtypeStruct((N, F), jnp.float32)] * 2,
    )(acc1[0], acc1[1], xlin1, dinv2d, b1p, W2p)

    acc2 = _sc_scatter(y2, src, dst, z128)

    gx = pl.pallas_call(
        _k3_body,
        grid=grid,
        in_specs=[_row_spec(F), _row_spec(F), _row_spec(F), _row_spec(1),
                  _full_spec((1, F)), _full_spec((F, F)), _full_spec((1, F))],
        out_specs=_row_spec(F),
        out_shape=jax.ShapeDtypeStruct((N, F), jnp.float32),
    )(acc2[0], acc2[1], xlin2, dinv2d, b2p, Wihp, gbias)

    out = pl.pallas_call(
        _lstm_body,
        out_shape=jax.ShapeDtypeStruct((1, 1), jnp.float32),
    )(gx, whh_big, wfc_pad, bfc2d)
    return out
